# baseline (device time: 8657 ns/iter reference)
import jax
import jax.numpy as jnp
from jax import lax
from jax.experimental import pallas as pl
from jax.experimental.pallas import tpu as pltpu

C = 64


def kernel(x, dest):
    m, n = x.shape
    nch = m // C
    dest_row = dest.reshape(1, m)

    def body(x_ref, drow_ref, out_ref, stage, send_sems, recv_sems):
        my_x = lax.axis_index("x")
        my_y = lax.axis_index("y")
        peer = (1 - my_x, my_y)

        barrier_sem = pltpu.get_barrier_semaphore()
        pl.semaphore_signal(barrier_sem, inc=1, device_id=peer,
                            device_id_type=pl.DeviceIdType.MESH)

        d_row = drow_ref[...]
        keep_row = d_row == my_x
        keep_bf = keep_row.astype(jnp.bfloat16)
        ia2 = lax.broadcasted_iota(jnp.int32, (m, m), 0)
        ib2 = lax.broadcasted_iota(jnp.int32, (m, m), 1)
        tri = (ia2 < ib2).astype(jnp.bfloat16)
        kpos = jnp.dot(
            keep_bf, tri, preferred_element_type=jnp.float32
        ).astype(jnp.int32)
        col = lax.broadcasted_iota(jnp.int32, (1, m), 1)
        spos = col - kpos
        n_keep = kpos[0, m - 1] + jnp.where(d_row[0, m - 1] == my_x, 1, 0)
        n_send = m - n_keep
        n_recv = n_send

        keep_base = jnp.where(my_x == 0, 0, n_recv)
        rb_peer = jnp.where(my_x == 0, 0, m - n_send)
        rb_al_peer = (rb_peer // 8) * 8
        pad = rb_peer - rb_al_peer
        padded = pad + n_send
        recv_base = jnp.where(my_x == 0, n_keep, 0)
        rb_al = (recv_base // 8) * 8
        padded_r = recv_base - rb_al + n_recv

        tgt_s = jnp.where(keep_row, -1, pad + spos)
        p_send = (ia2 == tgt_s).astype(jnp.bfloat16)
        x_bf = x_ref[...].astype(jnp.bfloat16)
        stage[...] = jnp.dot(
            p_send, x_bf, preferred_element_type=jnp.float32
        ).astype(jnp.bfloat16)

        pl.semaphore_wait(barrier_sem, 1)

        for c in range(nch):
            @pl.when(c * C < padded)
            def _(c=c):
                delta = jnp.maximum(0, rb_al_peer + (c + 1) * C - m)
                src_start = pl.multiple_of(c * C - delta, 8)
                dst_start = pl.multiple_of(rb_al_peer + c * C - delta, 8)
                rdma = pltpu.make_async_remote_copy(
                    src_ref=stage.at[pl.ds(src_start, C), :],
                    dst_ref=out_ref.at[pl.ds(dst_start, C), :],
                    send_sem=send_sems.at[c],
                    recv_sem=recv_sems.at[c],
                    device_id=peer,
                    device_id_type=pl.DeviceIdType.MESH,
                )
                rdma.start()

        tgt_k = jnp.where(keep_row, kpos + keep_base, -1)
        p_keep = (ia2 == tgt_k).astype(jnp.bfloat16)
        keep_part = jnp.dot(
            p_keep, x_bf, preferred_element_type=jnp.float32
        ).astype(jnp.bfloat16)
        nch_r = (padded_r + C - 1) // C
        hi = jnp.minimum(rb_al + nch_r * C, m)
        row = lax.broadcasted_iota(jnp.int32, (m, 1), 0)
        cover = (row >= rb_al) & (row < hi)

        for c in range(nch):
            @pl.when(c * C < padded_r)
            def _(c=c):
                rdma = pltpu.make_async_remote_copy(
                    src_ref=stage.at[pl.ds(0, C), :],
                    dst_ref=out_ref.at[pl.ds(0, C), :],
                    send_sem=send_sems.at[c],
                    recv_sem=recv_sems.at[c],
                    device_id=peer,
                    device_id_type=pl.DeviceIdType.MESH,
                )
                rdma.wait_recv()

        out_ref[...] = (
            jnp.where(cover, out_ref[...], jnp.bfloat16(0)) + keep_part
        )

        for c in range(nch):
            @pl.when(c * C < padded)
            def _(c=c):
                rdma = pltpu.make_async_remote_copy(
                    src_ref=stage.at[pl.ds(0, C), :],
                    dst_ref=out_ref.at[pl.ds(0, C), :],
                    send_sem=send_sems.at[c],
                    recv_sem=recv_sems.at[c],
                    device_id=peer,
                    device_id_type=pl.DeviceIdType.MESH,
                )
                rdma.wait_send()

    return pl.pallas_call(
        body,
        out_shape=jax.ShapeDtypeStruct((m, n), jnp.bfloat16),
        in_specs=[
            pl.BlockSpec(memory_space=pltpu.VMEM),
            pl.BlockSpec(memory_space=pltpu.VMEM),
        ],
        out_specs=pl.BlockSpec(memory_space=pltpu.VMEM),
        scratch_shapes=[
            pltpu.VMEM((m, n), jnp.bfloat16),
            pltpu.SemaphoreType.DMA((nch,)),
            pltpu.SemaphoreType.DMA((nch,)),
        ],
        compiler_params=pltpu.CompilerParams(collective_id=0),
    )(x, dest_row)


# device time: 8596 ns/iter; 1.0071x vs baseline; 1.0071x over previous
import jax
import jax.numpy as jnp
from jax import lax
from jax.experimental import pallas as pl
from jax.experimental.pallas import tpu as pltpu

C = 256


def kernel(x, dest):
    m, n = x.shape
    nch = m // C
    dest_row = dest.reshape(1, m)

    def body(x_ref, drow_ref, out_ref, stage, send_sems, recv_sems):
        my_x = lax.axis_index("x")
        my_y = lax.axis_index("y")
        peer = (1 - my_x, my_y)

        barrier_sem = pltpu.get_barrier_semaphore()
        pl.semaphore_signal(barrier_sem, inc=1, device_id=peer,
                            device_id_type=pl.DeviceIdType.MESH)

        d_row = drow_ref[...]
        keep_row = d_row == my_x
        keep_bf = keep_row.astype(jnp.bfloat16)
        ia2 = lax.broadcasted_iota(jnp.int32, (m, m), 0)
        ib2 = lax.broadcasted_iota(jnp.int32, (m, m), 1)
        tri = (ia2 < ib2).astype(jnp.bfloat16)
        kpos = jnp.dot(
            keep_bf, tri, preferred_element_type=jnp.float32
        ).astype(jnp.int32)
        col = lax.broadcasted_iota(jnp.int32, (1, m), 1)
        spos = col - kpos
        n_keep = kpos[0, m - 1] + jnp.where(d_row[0, m - 1] == my_x, 1, 0)
        n_send = m - n_keep
        n_recv = n_send

        keep_base = jnp.where(my_x == 0, 0, n_recv)
        rb_peer = jnp.where(my_x == 0, 0, m - n_send)
        rb_al_peer = (rb_peer // 8) * 8
        pad = rb_peer - rb_al_peer
        padded = pad + n_send
        recv_base = jnp.where(my_x == 0, n_keep, 0)
        rb_al = (recv_base // 8) * 8
        padded_r = recv_base - rb_al + n_recv

        tgt_s = jnp.where(keep_row, -1, pad + spos)
        p_send = (ia2 == tgt_s).astype(jnp.bfloat16)
        x_bf = x_ref[...].astype(jnp.bfloat16)
        stage[pl.ds(0, C), :] = jnp.zeros((C, n), jnp.bfloat16)
        stage[pl.ds(C, m), :] = jnp.dot(
            p_send, x_bf, preferred_element_type=jnp.float32
        ).astype(jnp.bfloat16)

        pl.semaphore_wait(barrier_sem, 1)

        for c in range(nch):
            @pl.when(c * C < padded)
            def _(c=c):
                delta = jnp.maximum(0, rb_al_peer + (c + 1) * C - m)
                src_start = pl.multiple_of(C + c * C - delta, 8)
                dst_start = pl.multiple_of(rb_al_peer + c * C - delta, 8)
                rdma = pltpu.make_async_remote_copy(
                    src_ref=stage.at[pl.ds(src_start, C), :],
                    dst_ref=out_ref.at[pl.ds(dst_start, C), :],
                    send_sem=send_sems.at[c],
                    recv_sem=recv_sems.at[c],
                    device_id=peer,
                    device_id_type=pl.DeviceIdType.MESH,
                )
                rdma.start()

        tgt_k = jnp.where(keep_row, kpos + keep_base, -1)
        p_keep = (ia2 == tgt_k).astype(jnp.bfloat16)
        keep_part = jnp.dot(
            p_keep, x_bf, preferred_element_type=jnp.float32
        ).astype(jnp.bfloat16)
        nch_r = (padded_r + C - 1) // C
        delta0 = jnp.maximum(0, rb_al + C - m)
        lo = jnp.where(padded_r > 0, rb_al - delta0, m)
        hi = jnp.minimum(rb_al + nch_r * C, m)
        row = lax.broadcasted_iota(jnp.int32, (m, 1), 0)
        cover = (row >= lo) & (row < hi)

        for c in range(nch):
            @pl.when(c * C < padded_r)
            def _(c=c):
                rdma = pltpu.make_async_remote_copy(
                    src_ref=stage.at[pl.ds(0, C), :],
                    dst_ref=out_ref.at[pl.ds(0, C), :],
                    send_sem=send_sems.at[c],
                    recv_sem=recv_sems.at[c],
                    device_id=peer,
                    device_id_type=pl.DeviceIdType.MESH,
                )
                rdma.wait_recv()

        out_ref[...] = (
            jnp.where(cover, out_ref[...], jnp.bfloat16(0)) + keep_part
        )

        for c in range(nch):
            @pl.when(c * C < padded)
            def _(c=c):
                rdma = pltpu.make_async_remote_copy(
                    src_ref=stage.at[pl.ds(0, C), :],
                    dst_ref=out_ref.at[pl.ds(0, C), :],
                    send_sem=send_sems.at[c],
                    recv_sem=recv_sems.at[c],
                    device_id=peer,
                    device_id_type=pl.DeviceIdType.MESH,
                )
                rdma.wait_send()

    return pl.pallas_call(
        body,
        out_shape=jax.ShapeDtypeStruct((m, n), jnp.bfloat16),
        in_specs=[
            pl.BlockSpec(memory_space=pltpu.VMEM),
            pl.BlockSpec(memory_space=pltpu.VMEM),
        ],
        out_specs=pl.BlockSpec(memory_space=pltpu.VMEM),
        scratch_shapes=[
            pltpu.VMEM((C + m, n), jnp.bfloat16),
            pltpu.SemaphoreType.DMA((nch,)),
            pltpu.SemaphoreType.DMA((nch,)),
        ],
        compiler_params=pltpu.CompilerParams(collective_id=0),
    )(x, dest_row)


# device time: 8122 ns/iter; 1.0659x vs baseline; 1.0584x over previous
import jax
import jax.numpy as jnp
from jax import lax
from jax.experimental import pallas as pl
from jax.experimental.pallas import tpu as pltpu

C = 128


def kernel(x, dest):
    m, n = x.shape
    nch = m // C
    dest_row = dest.reshape(1, m)

    def body(x_ref, drow_ref, out_ref, stage, send_sems, recv_sems):
        my_x = lax.axis_index("x")
        my_y = lax.axis_index("y")
        peer = (1 - my_x, my_y)

        barrier_sem = pltpu.get_barrier_semaphore()
        pl.semaphore_signal(barrier_sem, inc=1, device_id=peer,
                            device_id_type=pl.DeviceIdType.MESH)

        d_row = drow_ref[...]
        keep_row = d_row == my_x
        keep_bf = keep_row.astype(jnp.bfloat16)
        ia2 = lax.broadcasted_iota(jnp.int32, (m, m), 0)
        ib2 = lax.broadcasted_iota(jnp.int32, (m, m), 1)
        tri = (ia2 < ib2).astype(jnp.bfloat16)
        kpos = jnp.dot(
            keep_bf, tri, preferred_element_type=jnp.float32
        ).astype(jnp.int32)
        col = lax.broadcasted_iota(jnp.int32, (1, m), 1)
        spos = col - kpos
        n_keep = kpos[0, m - 1] + jnp.where(d_row[0, m - 1] == my_x, 1, 0)
        n_send = m - n_keep
        n_recv = n_send

        keep_base = jnp.where(my_x == 0, 0, n_recv)
        rb_peer = jnp.where(my_x == 0, 0, m - n_send)
        rb_al_peer = (rb_peer // 8) * 8
        pad = rb_peer - rb_al_peer
        padded = pad + n_send
        recv_base = jnp.where(my_x == 0, n_keep, 0)
        rb_al = (recv_base // 8) * 8
        padded_r = recv_base - rb_al + n_recv

        tgt_s = jnp.where(keep_row, -1, pad + spos)
        x_bf = x_ref[...].astype(jnp.bfloat16)
        stage[pl.ds(0, C), :] = jnp.zeros((C, n), jnp.bfloat16)
        iac = lax.broadcasted_iota(jnp.int32, (C, m), 0)

        barrier_waited = False
        for c in range(nch):
            @pl.when(c * C < padded)
            def _(c=c):
                p_c = ((iac + c * C) == tgt_s).astype(jnp.bfloat16)
                stage[pl.ds(C + c * C, C), :] = jnp.dot(
                    p_c, x_bf, preferred_element_type=jnp.float32
                ).astype(jnp.bfloat16)
            if not barrier_waited:
                pl.semaphore_wait(barrier_sem, 1)
                barrier_waited = True

            @pl.when(c * C < padded)
            def _(c=c):
                delta = jnp.maximum(0, rb_al_peer + (c + 1) * C - m)
                src_start = pl.multiple_of(C + c * C - delta, 8)
                dst_start = pl.multiple_of(rb_al_peer + c * C - delta, 8)
                rdma = pltpu.make_async_remote_copy(
                    src_ref=stage.at[pl.ds(src_start, C), :],
                    dst_ref=out_ref.at[pl.ds(dst_start, C), :],
                    send_sem=send_sems.at[c],
                    recv_sem=recv_sems.at[c],
                    device_id=peer,
                    device_id_type=pl.DeviceIdType.MESH,
                )
                rdma.start()

        tgt_k = jnp.where(keep_row, kpos + keep_base, -1)
        p_keep = (ia2 == tgt_k).astype(jnp.bfloat16)
        keep_part = jnp.dot(
            p_keep, x_bf, preferred_element_type=jnp.float32
        ).astype(jnp.bfloat16)
        nch_r = (padded_r + C - 1) // C
        delta0 = jnp.maximum(0, rb_al + C - m)
        lo = jnp.where(padded_r > 0, rb_al - delta0, m)
        hi = jnp.minimum(rb_al + nch_r * C, m)
        row = lax.broadcasted_iota(jnp.int32, (m, 1), 0)
        cover = (row >= lo) & (row < hi)

        for c in range(nch):
            @pl.when(c * C < padded_r)
            def _(c=c):
                rdma = pltpu.make_async_remote_copy(
                    src_ref=stage.at[pl.ds(0, C), :],
                    dst_ref=out_ref.at[pl.ds(0, C), :],
                    send_sem=send_sems.at[c],
                    recv_sem=recv_sems.at[c],
                    device_id=peer,
                    device_id_type=pl.DeviceIdType.MESH,
                )
                rdma.wait_recv()

        out_ref[...] = (
            jnp.where(cover, out_ref[...], jnp.bfloat16(0)) + keep_part
        )

        for c in range(nch):
            @pl.when(c * C < padded)
            def _(c=c):
                rdma = pltpu.make_async_remote_copy(
                    src_ref=stage.at[pl.ds(0, C), :],
                    dst_ref=out_ref.at[pl.ds(0, C), :],
                    send_sem=send_sems.at[c],
                    recv_sem=recv_sems.at[c],
                    device_id=peer,
                    device_id_type=pl.DeviceIdType.MESH,
                )
                rdma.wait_send()

    return pl.pallas_call(
        body,
        out_shape=jax.ShapeDtypeStruct((m, n), jnp.bfloat16),
        in_specs=[
            pl.BlockSpec(memory_space=pltpu.VMEM),
            pl.BlockSpec(memory_space=pltpu.VMEM),
        ],
        out_specs=pl.BlockSpec(memory_space=pltpu.VMEM),
        scratch_shapes=[
            pltpu.VMEM((C + m, n), jnp.bfloat16),
            pltpu.SemaphoreType.DMA((nch,)),
            pltpu.SemaphoreType.DMA((nch,)),
        ],
        compiler_params=pltpu.CompilerParams(collective_id=0),
    )(x, dest_row)
